# Initial kernel scaffold; baseline (speedup 1.0000x reference)
import jax
import jax.numpy as jnp
from jax.experimental import pallas as pl


def kernel(x, edge_index, edge_weight, W1, b1, W2, b2, Wn, bn, Wr, br):
    raise NotImplementedError("write your pallas kernel here")


# speedup vs baseline: 6.7672x; 6.7672x over previous
import functools

import jax
import jax.numpy as jnp
from jax import lax
from jax.experimental import pallas as pl
from jax.experimental.pallas import tpu as pltpu
from jax.experimental.pallas import tpu_sc as plsc

N = 10000
E = 320000
IN_CH = 128
HID = 256
HALF = 128

G = 128
NPAD = 10240
NGROUPS = E // G
NC = 2
NS = 16

_sc_mesh = plsc.VectorSubcoreMesh(
    core_axis_name="c", subcore_axis_name="s", num_cores=NC, num_subcores=NS
)


@functools.partial(
    pl.kernel,
    out_type=jax.ShapeDtypeStruct((NC, NPAD), jnp.float32),
    mesh=_sc_mesh,
    scratch_types=[
        pltpu.VMEM((G,), jnp.float32),
        pltpu.VMEM((1, G), jnp.int32),
        pltpu.VMEM((640,), jnp.float32),
        pltpu.VMEM_SHARED((NPAD,), jnp.float32),
    ],
)
def _deg_kernel(dst2_hbm, w_hbm, out_hbm, w_v, dst_v, zbuf, deg_sh):
    c = lax.axis_index("c")
    s = lax.axis_index("s")

    def zb(i, carry):
        zbuf[pl.ds(i * 16, 16)] = jnp.zeros((16,), jnp.float32)
        return carry

    lax.fori_loop(0, 40, zb, 0)

    pltpu.sync_copy(zbuf, deg_sh.at[pl.ds(s * 640, 640)])
    plsc.subcore_barrier()

    def grp(gi, carry):
        g_rel = s + gi * NS

        @pl.when(g_rel < NGROUPS // NC)
        def _():
            g = c * (NGROUPS // NC) + g_rel
            pltpu.sync_copy(w_hbm.at[pl.ds(g * G, G)], w_v)
            pltpu.sync_copy(dst2_hbm.at[pl.ds(g, 1)], dst_v)
            pltpu.sync_copy(w_v, deg_sh.at[dst_v.at[0]], add=True)

        return carry

    lax.fori_loop(0, (NGROUPS // NC + NS - 1) // NS, grp, 0)
    plsc.subcore_barrier()

    pltpu.sync_copy(deg_sh.at[pl.ds(s * 640, 640)], out_hbm.at[c, pl.ds(s * 640, 640)])


_ROWS_PER_TILE = NPAD // NS
_ZROWS = 128


@functools.partial(
    pl.kernel,
    out_type=[
        jax.ShapeDtypeStruct((NPAD, HALF), jnp.float32),
        jax.ShapeDtypeStruct((NPAD, HALF), jnp.float32),
    ],
    mesh=_sc_mesh,
    scratch_types=[
        pltpu.VMEM((G,), jnp.int32),
        pltpu.VMEM((1, G), jnp.int32),
        pltpu.VMEM((G,), jnp.float32),
        pltpu.VMEM((G, HALF), jnp.float32),
        pltpu.VMEM((_ZROWS, HALF), jnp.float32),
        pltpu.VMEM_SHARED((NPAD, HALF), jnp.float32),
        pltpu.SemaphoreType.DMA,
    ],
)
def _agg_kernel(hpA, hpB, src_hbm, dst2_hbm, w_hbm, outA, outB,
                src_v, dst_v, w_v, rows, zbuf, acc_sh, sem):
    c = lax.axis_index("c")
    s = lax.axis_index("s")

    def zrow(r, carry):
        for v8 in range(HALF // 16):
            zbuf[r, pl.ds(v8 * 16, 16)] = jnp.zeros((16,), jnp.float32)
        return carry

    lax.fori_loop(0, _ZROWS, zrow, 0)
    for k5 in range(_ROWS_PER_TILE // _ZROWS):
        pltpu.sync_copy(zbuf, acc_sh.at[pl.ds(s * _ROWS_PER_TILE + k5 * _ZROWS, _ZROWS)])
    plsc.subcore_barrier()

    def grp(gi, carry):
        g = s + gi * NS

        @pl.when(g < NGROUPS)
        def _():
            base = g * G
            pltpu.sync_copy(src_hbm.at[pl.ds(base, G)], src_v)
            pltpu.sync_copy(dst2_hbm.at[pl.ds(g, 1)], dst_v)
            pltpu.sync_copy(w_hbm.at[pl.ds(base, G)], w_v)

            @pl.when(c == 0)
            def _():
                pltpu.async_copy(hpA.at[src_v], rows, sem).wait()

            @pl.when(c == 1)
            def _():
                pltpu.async_copy(hpB.at[src_v], rows, sem).wait()

            def edge16(eo, carry2):
                wv16 = w_v[pl.ds(eo * 16, 16)]
                for lane in range(16):
                    sw = wv16[lane]
                    e = eo * 16 + lane
                    for v8 in range(HALF // 16):
                        rows[e, pl.ds(v8 * 16, 16)] = rows[e, pl.ds(v8 * 16, 16)] * sw
                return carry2

            lax.fori_loop(0, G // 16, edge16, 0)
            pltpu.sync_copy(rows, acc_sh.at[dst_v.at[0]], add=True)

        return carry

    lax.fori_loop(0, (NGROUPS + NS - 1) // NS, grp, 0)
    plsc.subcore_barrier()

    for k5 in range(_ROWS_PER_TILE // _ZROWS):
        r0 = s * _ROWS_PER_TILE + k5 * _ZROWS

        @pl.when(c == 0)
        def _():
            pltpu.sync_copy(acc_sh.at[pl.ds(r0, _ZROWS)], outA.at[pl.ds(r0, _ZROWS)])

        @pl.when(c == 1)
        def _():
            pltpu.sync_copy(acc_sh.at[pl.ds(r0, _ZROWS)], outB.at[pl.ds(r0, _ZROWS)])


_RB = 1000


def _k1_body(deg_ref, x_ref, w1_ref, hpa_ref, hpb_ref, dis_ref):
    dis = lax.rsqrt(deg_ref[...] + 1.0)
    h = jnp.dot(x_ref[...], w1_ref[...], preferred_element_type=jnp.float32)
    hp = h * dis
    hpa_ref[...] = hp[:, :HALF]
    hpb_ref[...] = hp[:, HALF:]
    dis_ref[...] = dis


def _tc_prep(deg_col, x, W1):
    grid = (N // _RB,)
    return pl.pallas_call(
        _k1_body,
        grid=grid,
        in_specs=[
            pl.BlockSpec((_RB, 1), lambda i: (i, 0)),
            pl.BlockSpec((_RB, IN_CH), lambda i: (i, 0)),
            pl.BlockSpec((IN_CH, HID), lambda i: (0, 0)),
        ],
        out_specs=[
            pl.BlockSpec((_RB, HALF), lambda i: (i, 0)),
            pl.BlockSpec((_RB, HALF), lambda i: (i, 0)),
            pl.BlockSpec((_RB, 1), lambda i: (i, 0)),
        ],
        out_shape=[
            jax.ShapeDtypeStruct((N, HALF), jnp.float32),
            jax.ShapeDtypeStruct((N, HALF), jnp.float32),
            jax.ShapeDtypeStruct((N, 1), jnp.float32),
        ],
    )(deg_col, x, W1)


def _k2_body(aa_ref, ab_ref, ha_ref, hb_ref, dis_ref, b_ref, w2_ref,
             oa_ref, ob_ref):
    dis = dis_ref[...]
    z = jnp.concatenate(
        [aa_ref[...] + ha_ref[...], ab_ref[...] + hb_ref[...]], axis=1)
    z = jnp.maximum(z * dis + b_ref[...], 0.0)
    h2 = jnp.dot(z, w2_ref[...], preferred_element_type=jnp.float32)
    hp2 = h2 * dis
    oa_ref[...] = hp2[:, :HALF]
    ob_ref[...] = hp2[:, HALF:]


def _tc_mid(accA, accB, hpA, hpB, dis, b1, W2):
    grid = (N // _RB,)
    row = lambda i: (i, 0)
    return pl.pallas_call(
        _k2_body,
        grid=grid,
        in_specs=[
            pl.BlockSpec((_RB, HALF), row),
            pl.BlockSpec((_RB, HALF), row),
            pl.BlockSpec((_RB, HALF), row),
            pl.BlockSpec((_RB, HALF), row),
            pl.BlockSpec((_RB, 1), row),
            pl.BlockSpec((1, HID), lambda i: (0, 0)),
            pl.BlockSpec((HID, HID), lambda i: (0, 0)),
        ],
        out_specs=[
            pl.BlockSpec((_RB, HALF), row),
            pl.BlockSpec((_RB, HALF), row),
        ],
        out_shape=[
            jax.ShapeDtypeStruct((N, HALF), jnp.float32),
            jax.ShapeDtypeStruct((N, HALF), jnp.float32),
        ],
    )(accA, accB, hpA, hpB, dis, b1, W2)


def _k3_body(aa_ref, ab_ref, ha_ref, hb_ref, dis_ref, b_ref, wr_ref, br_ref,
             h2_ref, resc_ref):
    dis = dis_ref[...]
    z = jnp.concatenate(
        [aa_ref[...] + ha_ref[...], ab_ref[...] + hb_ref[...]], axis=1)
    z = jnp.maximum(z * dis + b_ref[...], 0.0)
    h2_ref[...] = z
    r = jnp.dot(z, wr_ref[...], preferred_element_type=jnp.float32) + br_ref[...]
    resc_ref[...] = 1.0 / (1.0 + jnp.exp(-r))


def _tc_final(accA, accB, hpA, hpB, dis, b2, Wr, br):
    grid = (N // _RB,)
    row = lambda i: (i, 0)
    return pl.pallas_call(
        _k3_body,
        grid=grid,
        in_specs=[
            pl.BlockSpec((_RB, HALF), row),
            pl.BlockSpec((_RB, HALF), row),
            pl.BlockSpec((_RB, HALF), row),
            pl.BlockSpec((_RB, HALF), row),
            pl.BlockSpec((_RB, 1), row),
            pl.BlockSpec((1, HID), lambda i: (0, 0)),
            pl.BlockSpec((HID, 1), lambda i: (0, 0)),
            pl.BlockSpec((1, 1), lambda i: (0, 0)),
        ],
        out_specs=[
            pl.BlockSpec((_RB, HID), row),
            pl.BlockSpec((_RB, 1), row),
        ],
        out_shape=[
            jax.ShapeDtypeStruct((N, HID), jnp.float32),
            jax.ShapeDtypeStruct((N, 1), jnp.float32),
        ],
    )(accA, accB, hpA, hpB, dis, b2, Wr, br)


_HRB = 400


def _k4_body(h2_ref, wn_ref, bn_ref, logit_ref, m_ref, s_ref):
    i = pl.program_id(0)
    tile = jnp.dot(h2_ref[...], wn_ref[...], preferred_element_type=jnp.float32)
    tile = tile + bn_ref[...]
    logit_ref[...] = tile
    tmax = jnp.max(tile, axis=0, keepdims=True)

    @pl.when(i == 0)
    def _():
        m_ref[...] = tmax
        s_ref[...] = jnp.sum(jnp.exp(tile - tmax), axis=0, keepdims=True)

    @pl.when(i > 0)
    def _():
        m_old = m_ref[...]
        m_new = jnp.maximum(m_old, tmax)
        s_ref[...] = s_ref[...] * jnp.exp(m_old - m_new) + jnp.sum(
            jnp.exp(tile - m_new), axis=0, keepdims=True)
        m_ref[...] = m_new


def _tc_head(h2, Wn, bn_row):
    grid = (N // _HRB,)
    return pl.pallas_call(
        _k4_body,
        grid=grid,
        in_specs=[
            pl.BlockSpec((_HRB, HID), lambda i: (i, 0)),
            pl.BlockSpec((HID, N), lambda i: (0, 0)),
            pl.BlockSpec((1, N), lambda i: (0, 0)),
        ],
        out_specs=[
            pl.BlockSpec((_HRB, N), lambda i: (i, 0)),
            pl.BlockSpec((1, N), lambda i: (0, 0)),
            pl.BlockSpec((1, N), lambda i: (0, 0)),
        ],
        out_shape=[
            jax.ShapeDtypeStruct((N, N), jnp.float32),
            jax.ShapeDtypeStruct((1, N), jnp.float32),
            jax.ShapeDtypeStruct((1, N), jnp.float32),
        ],
    )(h2, Wn, bn_row)


_NRB = 200


def _k5_body(l_ref, m_ref, s_ref, out_ref):
    out_ref[...] = jnp.exp(l_ref[...] - m_ref[...]) * (1.0 / s_ref[...])


def _tc_norm(logits, m, s):
    grid = (N // _NRB,)
    return pl.pallas_call(
        _k5_body,
        grid=grid,
        in_specs=[
            pl.BlockSpec((_NRB, N), lambda i: (i, 0)),
            pl.BlockSpec((1, N), lambda i: (0, 0)),
            pl.BlockSpec((1, N), lambda i: (0, 0)),
        ],
        out_specs=pl.BlockSpec((_NRB, N), lambda i: (i, 0)),
        out_shape=jax.ShapeDtypeStruct((N, N), jnp.float32),
    )(logits, m, s)


def kernel(x, edge_index, edge_weight, W1, b1, W2, b2, Wn, bn, Wr, br):
    src = edge_index[0]
    dst = edge_index[1]
    dst2 = dst.reshape(NGROUPS, G)

    deg2 = _deg_kernel(dst2, edge_weight)
    deg_col = (deg2[0, :N] + deg2[1, :N]).reshape(N, 1)

    hpA, hpB, dis = _tc_prep(deg_col, x, W1)
    accA, accB = _agg_kernel(hpA, hpB, src, dst2, edge_weight)
    accA = accA[:N]
    accB = accB[:N]
    hp2A, hp2B = _tc_mid(accA, accB, hpA, hpB, dis, b1.reshape(1, HID), W2)
    acc2A, acc2B = _agg_kernel(hp2A, hp2B, src, dst2, edge_weight)
    acc2A = acc2A[:N]
    acc2B = acc2B[:N]
    h2, rescue = _tc_final(acc2A, acc2B, hp2A, hp2B, dis,
                           b2.reshape(1, HID), Wr, br.reshape(1, 1))

    logits, m, s = _tc_head(h2, Wn, bn.reshape(1, N))
    node_selector = _tc_norm(logits, m, s)
    return node_selector, rescue.reshape(N)


# speedup vs baseline: 12.4797x; 1.8441x over previous
import functools

import jax
import jax.numpy as jnp
from jax import lax
from jax.experimental import pallas as pl
from jax.experimental.pallas import tpu as pltpu
from jax.experimental.pallas import tpu_sc as plsc

N = 10000
E = 320000
IN_CH = 128
HID = 256
HALF = 128

G = 128
NPAD = 10240
NGROUPS = E // G
NC = 2
NS = 16

_sc_mesh = plsc.VectorSubcoreMesh(
    core_axis_name="c", subcore_axis_name="s", num_cores=NC, num_subcores=NS
)


@functools.partial(
    pl.kernel,
    out_type=jax.ShapeDtypeStruct((NC, NPAD), jnp.float32),
    mesh=_sc_mesh,
    scratch_types=[
        pltpu.VMEM((G,), jnp.float32),
        pltpu.VMEM((1, G), jnp.int32),
        pltpu.VMEM((640,), jnp.float32),
        pltpu.VMEM_SHARED((NPAD,), jnp.float32),
    ],
)
def _deg_kernel(dst2_hbm, w_hbm, out_hbm, w_v, dst_v, zbuf, deg_sh):
    c = lax.axis_index("c")
    s = lax.axis_index("s")

    def zb(i, carry):
        zbuf[pl.ds(i * 16, 16)] = jnp.zeros((16,), jnp.float32)
        return carry

    lax.fori_loop(0, 40, zb, 0)

    pltpu.sync_copy(zbuf, deg_sh.at[pl.ds(s * 640, 640)])
    plsc.subcore_barrier()

    def grp(gi, carry):
        g_rel = s + gi * NS

        @pl.when(g_rel < NGROUPS // NC)
        def _():
            g = c * (NGROUPS // NC) + g_rel
            pltpu.sync_copy(w_hbm.at[pl.ds(g * G, G)], w_v)
            pltpu.sync_copy(dst2_hbm.at[pl.ds(g, 1)], dst_v)
            pltpu.sync_copy(w_v, deg_sh.at[dst_v.at[0]], add=True)

        return carry

    lax.fori_loop(0, (NGROUPS // NC + NS - 1) // NS, grp, 0)
    plsc.subcore_barrier()

    pltpu.sync_copy(deg_sh.at[pl.ds(s * 640, 640)], out_hbm.at[c, pl.ds(s * 640, 640)])


_ROWS_PER_TILE = NPAD // NS
_ZROWS = 128
_GPT = NGROUPS // NS
_TAIL = NGROUPS - _GPT * NS
_CH = 26
_NCH = _GPT // _CH


@functools.partial(
    pl.kernel,
    out_type=[
        jax.ShapeDtypeStruct((NPAD, HALF), jnp.float32),
        jax.ShapeDtypeStruct((NPAD, HALF), jnp.float32),
    ],
    mesh=_sc_mesh,
    scratch_types=[
        pltpu.VMEM((_CH * G,), jnp.int32),
        pltpu.VMEM((_CH, 1, G), jnp.int32),
        pltpu.VMEM((_CH * G,), jnp.float32),
        pltpu.VMEM((G, HALF), jnp.float32),
        pltpu.VMEM((G, HALF), jnp.float32),
        pltpu.VMEM_SHARED((NPAD, HALF), jnp.float32),
        pltpu.SemaphoreType.DMA,
        pltpu.SemaphoreType.DMA,
    ],
)
def _agg_kernel(hp2n, src_hbm, dst3_hbm, w_hbm, outA, outB,
                src_loc, dst_loc, w_loc, rows0, rows1, acc_sh,
                sem0, sem1):
    c = lax.axis_index("c")
    s = lax.axis_index("s")

    def zrow(r, carry):
        for v8 in range(HALF // 16):
            rows0[r, pl.ds(v8 * 16, 16)] = jnp.zeros((16,), jnp.float32)
        return carry

    lax.fori_loop(0, _ZROWS, zrow, 0)
    for k5 in range(_ROWS_PER_TILE // _ZROWS):
        pltpu.sync_copy(rows0, acc_sh.at[pl.ds(s * _ROWS_PER_TILE + k5 * _ZROWS, _ZROWS)])
    plsc.subcore_barrier()

    def gather(g, buf, sem):
        off = pl.multiple_of(g * G, G)
        return pltpu.async_copy(
            hp2n.at[src_loc.at[pl.ds(off, G)]], buf, sem)

    def gwait(g, buf, sem):
        off = pl.multiple_of(g * G, G)
        pltpu.make_async_copy(
            hp2n.at[src_loc.at[pl.ds(off, G)]], buf, sem).wait()

    def scale_scatter(g, buf):
        def edge16(eo, carry2):
            wv16 = w_loc[pl.ds(pl.multiple_of(g * G + eo * 16, 16), 16)]
            for lane in range(16):
                sw = wv16[lane]
                e = eo * 16 + lane
                for v8 in range(HALF // 16):
                    buf[e, pl.ds(v8 * 16, 16)] = buf[e, pl.ds(v8 * 16, 16)] * sw
            return carry2

        lax.fori_loop(0, G // 16, edge16, 0)
        pltpu.sync_copy(buf, acc_sh.at[dst_loc.at[g, 0]], add=True)

    def stage(k):
        ebase = pl.multiple_of(s * (_GPT * G) + k * (_CH * G), G)
        pltpu.sync_copy(src_hbm.at[pl.ds(ebase, _CH * G)], src_loc)
        pltpu.sync_copy(dst3_hbm.at[pl.ds(s * _GPT + k * _CH, _CH)], dst_loc)
        pltpu.sync_copy(w_hbm.at[pl.ds(ebase, _CH * G)], w_loc)

        def tf(i, carry):
            v = src_loc[pl.ds(i * 16, 16)]
            src_loc[pl.ds(i * 16, 16)] = v * 2 + c
            return carry

        lax.fori_loop(0, (_CH * G) // 16, tf, 0)

    def chunk(k, carry):
        stage(k)
        gather(0, rows0, sem0)

        def pair(p, carry2):
            g0 = 2 * p
            g1 = g0 + 1
            gather(g1, rows1, sem1)
            gwait(g0, rows0, sem0)
            scale_scatter(g0, rows0)

            @pl.when(p < _CH // 2 - 1)
            def _():
                gather(g0 + 2, rows0, sem0)

            gwait(g1, rows1, sem1)
            scale_scatter(g1, rows1)
            return carry2

        lax.fori_loop(0, _CH // 2, pair, 0)
        return carry

    lax.fori_loop(0, _NCH, chunk, 0)

    @pl.when(s < _TAIL)
    def _():
        gt = NS * _GPT + s
        pltpu.sync_copy(src_hbm.at[pl.ds(gt * G, G)], src_loc.at[pl.ds(0, G)])
        pltpu.sync_copy(dst3_hbm.at[pl.ds(gt, 1)], dst_loc.at[pl.ds(0, 1)])
        pltpu.sync_copy(w_hbm.at[pl.ds(gt * G, G)], w_loc.at[pl.ds(0, G)])

        def tf2(i, carry):
            v = src_loc[pl.ds(i * 16, 16)]
            src_loc[pl.ds(i * 16, 16)] = v * 2 + c
            return carry

        lax.fori_loop(0, G // 16, tf2, 0)
        gather(0, rows0, sem0)
        gwait(0, rows0, sem0)
        scale_scatter(0, rows0)

    plsc.subcore_barrier()

    for k5 in range(_ROWS_PER_TILE // _ZROWS):
        r0 = s * _ROWS_PER_TILE + k5 * _ZROWS

        @pl.when(c == 0)
        def _():
            pltpu.sync_copy(acc_sh.at[pl.ds(r0, _ZROWS)], outA.at[pl.ds(r0, _ZROWS)])

        @pl.when(c == 1)
        def _():
            pltpu.sync_copy(acc_sh.at[pl.ds(r0, _ZROWS)], outB.at[pl.ds(r0, _ZROWS)])


_RB = 1000


def _k1_body(deg_ref, x_ref, w1_ref, hp_ref, dis_ref):
    dis = lax.rsqrt(deg_ref[...] + 1.0)
    h = jnp.dot(x_ref[...], w1_ref[...], preferred_element_type=jnp.float32)
    hp_ref[...] = h * dis
    dis_ref[...] = dis


def _tc_prep(deg_col, x, W1):
    grid = (N // _RB,)
    return pl.pallas_call(
        _k1_body,
        grid=grid,
        in_specs=[
            pl.BlockSpec((_RB, 1), lambda i: (i, 0)),
            pl.BlockSpec((_RB, IN_CH), lambda i: (i, 0)),
            pl.BlockSpec((IN_CH, HID), lambda i: (0, 0)),
        ],
        out_specs=[
            pl.BlockSpec((_RB, HID), lambda i: (i, 0)),
            pl.BlockSpec((_RB, 1), lambda i: (i, 0)),
        ],
        out_shape=[
            jax.ShapeDtypeStruct((N, HID), jnp.float32),
            jax.ShapeDtypeStruct((N, 1), jnp.float32),
        ],
    )(deg_col, x, W1)


def _k2_body(aa_ref, ab_ref, hp_ref, dis_ref, b_ref, w2_ref, o_ref):
    dis = dis_ref[...]
    z = jnp.concatenate([aa_ref[...], ab_ref[...]], axis=1) + hp_ref[...]
    z = jnp.maximum(z * dis + b_ref[...], 0.0)
    h2 = jnp.dot(z, w2_ref[...], preferred_element_type=jnp.float32)
    o_ref[...] = h2 * dis


def _tc_mid(accA, accB, hp, dis, b1, W2):
    grid = (N // _RB,)
    row = lambda i: (i, 0)
    return pl.pallas_call(
        _k2_body,
        grid=grid,
        in_specs=[
            pl.BlockSpec((_RB, HALF), row),
            pl.BlockSpec((_RB, HALF), row),
            pl.BlockSpec((_RB, HID), row),
            pl.BlockSpec((_RB, 1), row),
            pl.BlockSpec((1, HID), lambda i: (0, 0)),
            pl.BlockSpec((HID, HID), lambda i: (0, 0)),
        ],
        out_specs=pl.BlockSpec((_RB, HID), row),
        out_shape=jax.ShapeDtypeStruct((N, HID), jnp.float32),
    )(accA, accB, hp, dis, b1, W2)


def _k3_body(aa_ref, ab_ref, hp_ref, dis_ref, b_ref, wr_ref, br_ref,
             h2_ref, resc_ref):
    dis = dis_ref[...]
    z = jnp.concatenate([aa_ref[...], ab_ref[...]], axis=1) + hp_ref[...]
    z = jnp.maximum(z * dis + b_ref[...], 0.0)
    h2_ref[...] = z
    r = jnp.dot(z, wr_ref[...], preferred_element_type=jnp.float32) + br_ref[...]
    resc_ref[...] = 1.0 / (1.0 + jnp.exp(-r))


def _tc_final(accA, accB, hp2, dis, b2, Wr, br):
    grid = (N // _RB,)
    row = lambda i: (i, 0)
    return pl.pallas_call(
        _k3_body,
        grid=grid,
        in_specs=[
            pl.BlockSpec((_RB, HALF), row),
            pl.BlockSpec((_RB, HALF), row),
            pl.BlockSpec((_RB, HID), row),
            pl.BlockSpec((_RB, 1), row),
            pl.BlockSpec((1, HID), lambda i: (0, 0)),
            pl.BlockSpec((HID, 1), lambda i: (0, 0)),
            pl.BlockSpec((1, 1), lambda i: (0, 0)),
        ],
        out_specs=[
            pl.BlockSpec((_RB, HID), row),
            pl.BlockSpec((_RB, 1), row),
        ],
        out_shape=[
            jax.ShapeDtypeStruct((N, HID), jnp.float32),
            jax.ShapeDtypeStruct((N, 1), jnp.float32),
        ],
    )(accA, accB, hp2, dis, b2, Wr, br)


_HRB = 200


def _k4_body(h2_ref, wn_ref, m_ref, s_ref):
    i = pl.program_id(0)
    tile = jnp.dot(h2_ref[...], wn_ref[...], preferred_element_type=jnp.float32)
    tmax = jnp.max(tile, axis=0, keepdims=True)

    @pl.when(i == 0)
    def _():
        m_ref[...] = tmax
        s_ref[...] = jnp.sum(jnp.exp(tile - tmax), axis=0, keepdims=True)

    @pl.when(i > 0)
    def _():
        m_old = m_ref[...]
        m_new = jnp.maximum(m_old, tmax)
        s_ref[...] = s_ref[...] * jnp.exp(m_old - m_new) + jnp.sum(
            jnp.exp(tile - m_new), axis=0, keepdims=True)
        m_ref[...] = m_new


def _tc_head_stats(h2, Wn):
    grid = (N // _HRB,)
    return pl.pallas_call(
        _k4_body,
        grid=grid,
        in_specs=[
            pl.BlockSpec((_HRB, HID), lambda i: (i, 0)),
            pl.BlockSpec((HID, N), lambda i: (0, 0)),
        ],
        out_specs=[
            pl.BlockSpec((1, N), lambda i: (0, 0)),
            pl.BlockSpec((1, N), lambda i: (0, 0)),
        ],
        out_shape=[
            jax.ShapeDtypeStruct((1, N), jnp.float32),
            jax.ShapeDtypeStruct((1, N), jnp.float32),
        ],
    )(h2, Wn)


def _k5_body(h2_ref, wn_ref, m_ref, s_ref, out_ref):
    tile = jnp.dot(h2_ref[...], wn_ref[...], preferred_element_type=jnp.float32)
    out_ref[...] = jnp.exp(tile - m_ref[...]) * (1.0 / s_ref[...])


def _tc_norm(h2, Wn, m, s):
    grid = (N // _HRB,)
    return pl.pallas_call(
        _k5_body,
        grid=grid,
        in_specs=[
            pl.BlockSpec((_HRB, HID), lambda i: (i, 0)),
            pl.BlockSpec((HID, N), lambda i: (0, 0)),
            pl.BlockSpec((1, N), lambda i: (0, 0)),
            pl.BlockSpec((1, N), lambda i: (0, 0)),
        ],
        out_specs=pl.BlockSpec((_HRB, N), lambda i: (i, 0)),
        out_shape=jax.ShapeDtypeStruct((N, N), jnp.float32),
    )(h2, Wn, m, s)


def kernel(x, edge_index, edge_weight, W1, b1, W2, b2, Wn, bn, Wr, br):
    src = edge_index[0]
    dst = edge_index[1]
    dst2 = dst.reshape(NGROUPS, G)
    dst3 = dst.reshape(NGROUPS, 1, G)

    deg2 = _deg_kernel(dst2, edge_weight)
    deg_col = (deg2[0, :N] + deg2[1, :N]).reshape(N, 1)

    hp, dis = _tc_prep(deg_col, x, W1)
    accA, accB = _agg_kernel(hp.reshape(2 * N, HALF), src, dst3, edge_weight)
    accA = accA[:N]
    accB = accB[:N]
    hp2 = _tc_mid(accA, accB, hp, dis, b1.reshape(1, HID), W2)
    acc2A, acc2B = _agg_kernel(hp2.reshape(2 * N, HALF), src, dst3, edge_weight)
    acc2A = acc2A[:N]
    acc2B = acc2B[:N]
    h2, rescue = _tc_final(acc2A, acc2B, hp2, dis,
                           b2.reshape(1, HID), Wr, br.reshape(1, 1))

    m, s = _tc_head_stats(h2, Wn)
    node_selector = _tc_norm(h2, Wn, m, s)
    return node_selector, rescue.reshape(N)


# speedup vs baseline: 12.7056x; 1.0181x over previous
import functools

import jax
import jax.numpy as jnp
from jax import lax
from jax.experimental import pallas as pl
from jax.experimental.pallas import tpu as pltpu
from jax.experimental.pallas import tpu_sc as plsc

N = 10000
E = 320000
IN_CH = 128
HID = 256
HALF = 128

G = 128
NPAD = 10240
NGROUPS = E // G
NC = 2
NS = 16

_sc_mesh = plsc.VectorSubcoreMesh(
    core_axis_name="c", subcore_axis_name="s", num_cores=NC, num_subcores=NS
)


_DGPT = NGROUPS // (NC * NS)
_DTAIL = NGROUPS - _DGPT * NC * NS


@functools.partial(
    pl.kernel,
    out_type=jax.ShapeDtypeStruct((NC, NPAD), jnp.float32),
    mesh=_sc_mesh,
    scratch_types=[
        pltpu.VMEM((_DGPT * G,), jnp.float32),
        pltpu.VMEM((_DGPT, 1, G), jnp.int32),
        pltpu.VMEM((640,), jnp.float32),
        pltpu.VMEM_SHARED((NPAD,), jnp.float32),
        pltpu.SemaphoreType.DMA,
    ],
)
def _deg_kernel(dst3_hbm, w_hbm, out_hbm, w_loc, dst_loc, zbuf, deg_sh, sem):
    c = lax.axis_index("c")
    s = lax.axis_index("s")
    t = c * NS + s

    def zb(i, carry):
        zbuf[pl.ds(i * 16, 16)] = jnp.zeros((16,), jnp.float32)
        return carry

    lax.fori_loop(0, 40, zb, 0)
    pltpu.sync_copy(zbuf, deg_sh.at[pl.ds(s * 640, 640)])

    ebase = pl.multiple_of(t * (_DGPT * G), G)
    pltpu.sync_copy(w_hbm.at[pl.ds(ebase, _DGPT * G)], w_loc)
    pltpu.sync_copy(dst3_hbm.at[pl.ds(t * _DGPT, _DGPT)], dst_loc)
    plsc.subcore_barrier()

    def fire(g, carry):
        off = pl.multiple_of(g * G, G)
        pltpu.async_copy(
            w_loc.at[pl.ds(off, G)], deg_sh.at[dst_loc.at[g, 0]], sem, add=True)
        return carry

    lax.fori_loop(0, _DGPT, fire, 0)

    def drain(g, carry):
        off = pl.multiple_of(g * G, G)
        pltpu.make_async_copy(
            w_loc.at[pl.ds(off, G)], deg_sh.at[dst_loc.at[g, 0]], sem).wait()
        return carry

    lax.fori_loop(0, _DGPT, drain, 0)

    @pl.when(t < _DTAIL)
    def _():
        gt = NC * NS * _DGPT + t
        pltpu.sync_copy(w_hbm.at[pl.ds(gt * G, G)], w_loc.at[pl.ds(0, G)])
        pltpu.sync_copy(dst3_hbm.at[pl.ds(gt, 1)], dst_loc.at[pl.ds(0, 1)])
        pltpu.async_copy(
            w_loc.at[pl.ds(0, G)], deg_sh.at[dst_loc.at[0, 0]], sem, add=True)
        pltpu.make_async_copy(
            w_loc.at[pl.ds(0, G)], deg_sh.at[dst_loc.at[0, 0]], sem).wait()

    plsc.subcore_barrier()
    pltpu.sync_copy(deg_sh.at[pl.ds(s * 640, 640)], out_hbm.at[c, pl.ds(s * 640, 640)])


_ROWS_PER_TILE = NPAD // NS
_ZROWS = 128
_GPT = NGROUPS // NS
_TAIL = NGROUPS - _GPT * NS
_CH = 26
_NCH = _GPT // _CH


@functools.partial(
    pl.kernel,
    out_type=[
        jax.ShapeDtypeStruct((NPAD, HALF), jnp.float32),
        jax.ShapeDtypeStruct((NPAD, HALF), jnp.float32),
    ],
    mesh=_sc_mesh,
    scratch_types=[
        pltpu.VMEM((_CH * G,), jnp.int32),
        pltpu.VMEM((_CH, 1, G), jnp.int32),
        pltpu.VMEM((_CH * G,), jnp.float32),
        pltpu.VMEM((G, HALF), jnp.float32),
        pltpu.VMEM((G, HALF), jnp.float32),
        pltpu.VMEM_SHARED((NPAD, HALF), jnp.float32),
        pltpu.SemaphoreType.DMA,
        pltpu.SemaphoreType.DMA,
        pltpu.SemaphoreType.DMA,
        pltpu.SemaphoreType.DMA,
    ],
)
def _agg_kernel(hp2n, src_hbm, dst3_hbm, w_hbm, outA, outB,
                src_loc, dst_loc, w_loc, rows0, rows1, acc_sh,
                sem0, sem1, sem2, sem3):
    c = lax.axis_index("c")
    s = lax.axis_index("s")

    def zrow(r, carry):
        for v8 in range(HALF // 16):
            rows0[r, pl.ds(v8 * 16, 16)] = jnp.zeros((16,), jnp.float32)
        return carry

    lax.fori_loop(0, _ZROWS, zrow, 0)
    for k5 in range(_ROWS_PER_TILE // _ZROWS):
        pltpu.sync_copy(rows0, acc_sh.at[pl.ds(s * _ROWS_PER_TILE + k5 * _ZROWS, _ZROWS)])
    plsc.subcore_barrier()

    def gather(g, buf, sem):
        off = pl.multiple_of(g * G, G)
        return pltpu.async_copy(
            hp2n.at[src_loc.at[pl.ds(off, G)]], buf, sem)

    def gwait(g, buf, sem):
        off = pl.multiple_of(g * G, G)
        pltpu.make_async_copy(
            hp2n.at[src_loc.at[pl.ds(off, G)]], buf, sem).wait()

    def scale(g, buf):
        def edge16(eo, carry2):
            wv16 = w_loc[pl.ds(pl.multiple_of(g * G + eo * 16, 16), 16)]
            for lane in range(16):
                sw = wv16[lane]
                e = eo * 16 + lane
                for v8 in range(HALF // 16):
                    buf[e, pl.ds(v8 * 16, 16)] = buf[e, pl.ds(v8 * 16, 16)] * sw
            return carry2

        lax.fori_loop(0, G // 16, edge16, 0)

    def scat_start(g, buf, sem):
        pltpu.async_copy(buf, acc_sh.at[dst_loc.at[g, 0]], sem, add=True)

    def scat_wait(g, buf, sem):
        pltpu.make_async_copy(buf, acc_sh.at[dst_loc.at[g, 0]], sem).wait()

    def stage(k):
        ebase = pl.multiple_of(s * (_GPT * G) + k * (_CH * G), G)
        pltpu.sync_copy(src_hbm.at[pl.ds(ebase, _CH * G)], src_loc)
        pltpu.sync_copy(dst3_hbm.at[pl.ds(s * _GPT + k * _CH, _CH)], dst_loc)
        pltpu.sync_copy(w_hbm.at[pl.ds(ebase, _CH * G)], w_loc)

        def tf(i, carry):
            v = src_loc[pl.ds(i * 16, 16)]
            src_loc[pl.ds(i * 16, 16)] = v * 2 + c
            return carry

        lax.fori_loop(0, (_CH * G) // 16, tf, 0)

    def chunk(k, carry):
        stage(k)
        gather(0, rows0, sem0)
        gather(1, rows1, sem1)

        def pair(p, carry2):
            g0 = 2 * p
            g1 = g0 + 1
            gwait(g0, rows0, sem0)
            scale(g0, rows0)
            scat_start(g0, rows0, sem2)
            gwait(g1, rows1, sem1)
            scale(g1, rows1)
            scat_start(g1, rows1, sem3)

            @pl.when(p < _CH // 2 - 1)
            def _():
                scat_wait(g0, rows0, sem2)
                gather(g0 + 2, rows0, sem0)
                scat_wait(g1, rows1, sem3)
                gather(g1 + 3 - 1, rows1, sem1)

            @pl.when(p == _CH // 2 - 1)
            def _():
                scat_wait(g0, rows0, sem2)
                scat_wait(g1, rows1, sem3)

            return carry2

        lax.fori_loop(0, _CH // 2, pair, 0)
        return carry

    lax.fori_loop(0, _NCH, chunk, 0)

    @pl.when(s < _TAIL)
    def _():
        gt = NS * _GPT + s
        pltpu.sync_copy(src_hbm.at[pl.ds(gt * G, G)], src_loc.at[pl.ds(0, G)])
        pltpu.sync_copy(dst3_hbm.at[pl.ds(gt, 1)], dst_loc.at[pl.ds(0, 1)])
        pltpu.sync_copy(w_hbm.at[pl.ds(gt * G, G)], w_loc.at[pl.ds(0, G)])

        def tf2(i, carry):
            v = src_loc[pl.ds(i * 16, 16)]
            src_loc[pl.ds(i * 16, 16)] = v * 2 + c
            return carry

        lax.fori_loop(0, G // 16, tf2, 0)
        gather(0, rows0, sem0)
        gwait(0, rows0, sem0)
        scale(0, rows0)
        scat_start(0, rows0, sem2)
        scat_wait(0, rows0, sem2)

    plsc.subcore_barrier()

    for k5 in range(_ROWS_PER_TILE // _ZROWS):
        r0 = s * _ROWS_PER_TILE + k5 * _ZROWS

        @pl.when(c == 0)
        def _():
            pltpu.sync_copy(acc_sh.at[pl.ds(r0, _ZROWS)], outA.at[pl.ds(r0, _ZROWS)])

        @pl.when(c == 1)
        def _():
            pltpu.sync_copy(acc_sh.at[pl.ds(r0, _ZROWS)], outB.at[pl.ds(r0, _ZROWS)])


_RB = 1000


def _k1_body(deg_ref, x_ref, w1_ref, hp_ref, dis_ref):
    dis = lax.rsqrt(deg_ref[...] + 1.0)
    h = jnp.dot(x_ref[...], w1_ref[...], preferred_element_type=jnp.float32)
    hp_ref[...] = h * dis
    dis_ref[...] = dis


def _tc_prep(deg_col, x, W1):
    grid = (N // _RB,)
    return pl.pallas_call(
        _k1_body,
        grid=grid,
        in_specs=[
            pl.BlockSpec((_RB, 1), lambda i: (i, 0)),
            pl.BlockSpec((_RB, IN_CH), lambda i: (i, 0)),
            pl.BlockSpec((IN_CH, HID), lambda i: (0, 0)),
        ],
        out_specs=[
            pl.BlockSpec((_RB, HID), lambda i: (i, 0)),
            pl.BlockSpec((_RB, 1), lambda i: (i, 0)),
        ],
        out_shape=[
            jax.ShapeDtypeStruct((N, HID), jnp.float32),
            jax.ShapeDtypeStruct((N, 1), jnp.float32),
        ],
    )(deg_col, x, W1)


def _k2_body(aa_ref, ab_ref, hp_ref, dis_ref, b_ref, w2_ref, o_ref):
    dis = dis_ref[...]
    z = jnp.concatenate([aa_ref[...], ab_ref[...]], axis=1) + hp_ref[...]
    z = jnp.maximum(z * dis + b_ref[...], 0.0)
    h2 = jnp.dot(z, w2_ref[...], preferred_element_type=jnp.float32)
    o_ref[...] = h2 * dis


def _tc_mid(accA, accB, hp, dis, b1, W2):
    grid = (N // _RB,)
    row = lambda i: (i, 0)
    return pl.pallas_call(
        _k2_body,
        grid=grid,
        in_specs=[
            pl.BlockSpec((_RB, HALF), row),
            pl.BlockSpec((_RB, HALF), row),
            pl.BlockSpec((_RB, HID), row),
            pl.BlockSpec((_RB, 1), row),
            pl.BlockSpec((1, HID), lambda i: (0, 0)),
            pl.BlockSpec((HID, HID), lambda i: (0, 0)),
        ],
        out_specs=pl.BlockSpec((_RB, HID), row),
        out_shape=jax.ShapeDtypeStruct((N, HID), jnp.float32),
    )(accA, accB, hp, dis, b1, W2)


def _k3_body(aa_ref, ab_ref, hp_ref, dis_ref, b_ref, wr_ref, br_ref,
             h2_ref, resc_ref):
    dis = dis_ref[...]
    z = jnp.concatenate([aa_ref[...], ab_ref[...]], axis=1) + hp_ref[...]
    z = jnp.maximum(z * dis + b_ref[...], 0.0)
    h2_ref[...] = z
    r = jnp.dot(z, wr_ref[...], preferred_element_type=jnp.float32) + br_ref[...]
    resc_ref[...] = 1.0 / (1.0 + jnp.exp(-r))


def _tc_final(accA, accB, hp2, dis, b2, Wr, br):
    grid = (N // _RB,)
    row = lambda i: (i, 0)
    return pl.pallas_call(
        _k3_body,
        grid=grid,
        in_specs=[
            pl.BlockSpec((_RB, HALF), row),
            pl.BlockSpec((_RB, HALF), row),
            pl.BlockSpec((_RB, HID), row),
            pl.BlockSpec((_RB, 1), row),
            pl.BlockSpec((1, HID), lambda i: (0, 0)),
            pl.BlockSpec((HID, 1), lambda i: (0, 0)),
            pl.BlockSpec((1, 1), lambda i: (0, 0)),
        ],
        out_specs=[
            pl.BlockSpec((_RB, HID), row),
            pl.BlockSpec((_RB, 1), row),
        ],
        out_shape=[
            jax.ShapeDtypeStruct((N, HID), jnp.float32),
            jax.ShapeDtypeStruct((N, 1), jnp.float32),
        ],
    )(accA, accB, hp2, dis, b2, Wr, br)


_HRB = 200


def _k4_body(h2_ref, wn_ref, m_ref, s_ref):
    i = pl.program_id(0)
    tile = jnp.dot(h2_ref[...], wn_ref[...], preferred_element_type=jnp.float32)
    tmax = jnp.max(tile, axis=0, keepdims=True)

    @pl.when(i == 0)
    def _():
        m_ref[...] = tmax
        s_ref[...] = jnp.sum(jnp.exp(tile - tmax), axis=0, keepdims=True)

    @pl.when(i > 0)
    def _():
        m_old = m_ref[...]
        m_new = jnp.maximum(m_old, tmax)
        s_ref[...] = s_ref[...] * jnp.exp(m_old - m_new) + jnp.sum(
            jnp.exp(tile - m_new), axis=0, keepdims=True)
        m_ref[...] = m_new


def _tc_head_stats(h2, Wn):
    grid = (N // _HRB,)
    return pl.pallas_call(
        _k4_body,
        grid=grid,
        in_specs=[
            pl.BlockSpec((_HRB, HID), lambda i: (i, 0)),
            pl.BlockSpec((HID, N), lambda i: (0, 0)),
        ],
        out_specs=[
            pl.BlockSpec((1, N), lambda i: (0, 0)),
            pl.BlockSpec((1, N), lambda i: (0, 0)),
        ],
        out_shape=[
            jax.ShapeDtypeStruct((1, N), jnp.float32),
            jax.ShapeDtypeStruct((1, N), jnp.float32),
        ],
    )(h2, Wn)


def _k5_body(h2_ref, wn_ref, m_ref, s_ref, out_ref):
    tile = jnp.dot(h2_ref[...], wn_ref[...], preferred_element_type=jnp.float32)
    out_ref[...] = jnp.exp(tile - m_ref[...]) * (1.0 / s_ref[...])


def _tc_norm(h2, Wn, m, s):
    grid = (N // _HRB,)
    return pl.pallas_call(
        _k5_body,
        grid=grid,
        in_specs=[
            pl.BlockSpec((_HRB, HID), lambda i: (i, 0)),
            pl.BlockSpec((HID, N), lambda i: (0, 0)),
            pl.BlockSpec((1, N), lambda i: (0, 0)),
            pl.BlockSpec((1, N), lambda i: (0, 0)),
        ],
        out_specs=pl.BlockSpec((_HRB, N), lambda i: (i, 0)),
        out_shape=jax.ShapeDtypeStruct((N, N), jnp.float32),
    )(h2, Wn, m, s)


def kernel(x, edge_index, edge_weight, W1, b1, W2, b2, Wn, bn, Wr, br):
    src = edge_index[0]
    dst = edge_index[1]
    dst3 = dst.reshape(NGROUPS, 1, G)

    deg2 = _deg_kernel(dst3, edge_weight)
    deg_col = (deg2[0, :N] + deg2[1, :N]).reshape(N, 1)

    hp, dis = _tc_prep(deg_col, x, W1)
    accA, accB = _agg_kernel(hp.reshape(2 * N, HALF), src, dst3, edge_weight)
    accA = accA[:N]
    accB = accB[:N]
    hp2 = _tc_mid(accA, accB, hp, dis, b1.reshape(1, HID), W2)
    acc2A, acc2B = _agg_kernel(hp2.reshape(2 * N, HALF), src, dst3, edge_weight)
    acc2A = acc2A[:N]
    acc2B = acc2B[:N]
    h2, rescue = _tc_final(acc2A, acc2B, hp2, dis,
                           b2.reshape(1, HID), Wr, br.reshape(1, 1))

    m, s = _tc_head_stats(h2, Wn)
    node_selector = _tc_norm(h2, Wn, m, s)
    return node_selector, rescue.reshape(N)


# speedup vs baseline: 13.8937x; 1.0935x over previous
import functools

import jax
import jax.numpy as jnp
from jax import lax
from jax.experimental import pallas as pl
from jax.experimental.pallas import tpu as pltpu
from jax.experimental.pallas import tpu_sc as plsc

N = 10000
E = 320000
IN_CH = 128
HID = 256
HALF = 128

G = 128
NPAD = 10240
NGROUPS = E // G
NC = 2
NS = 16

_sc_mesh = plsc.VectorSubcoreMesh(
    core_axis_name="c", subcore_axis_name="s", num_cores=NC, num_subcores=NS
)


_DGPT = NGROUPS // (NC * NS)
_DTAIL = NGROUPS - _DGPT * NC * NS


@functools.partial(
    pl.kernel,
    out_type=jax.ShapeDtypeStruct((NC, NPAD), jnp.float32),
    mesh=_sc_mesh,
    scratch_types=[
        pltpu.VMEM((_DGPT * G,), jnp.float32),
        pltpu.VMEM((_DGPT, 1, G), jnp.int32),
        pltpu.VMEM((640,), jnp.float32),
        pltpu.VMEM_SHARED((NPAD,), jnp.float32),
        pltpu.SemaphoreType.DMA,
    ],
)
def _deg_kernel(dst3_hbm, w_hbm, out_hbm, w_loc, dst_loc, zbuf, deg_sh, sem):
    c = lax.axis_index("c")
    s = lax.axis_index("s")
    t = c * NS + s

    def zb(i, carry):
        zbuf[pl.ds(i * 16, 16)] = jnp.zeros((16,), jnp.float32)
        return carry

    lax.fori_loop(0, 40, zb, 0)
    pltpu.sync_copy(zbuf, deg_sh.at[pl.ds(s * 640, 640)])

    ebase = pl.multiple_of(t * (_DGPT * G), G)
    pltpu.sync_copy(w_hbm.at[pl.ds(ebase, _DGPT * G)], w_loc)
    pltpu.sync_copy(dst3_hbm.at[pl.ds(t * _DGPT, _DGPT)], dst_loc)
    plsc.subcore_barrier()

    def fire(g, carry):
        off = pl.multiple_of(g * G, G)
        pltpu.async_copy(
            w_loc.at[pl.ds(off, G)], deg_sh.at[dst_loc.at[g, 0]], sem, add=True)
        return carry

    lax.fori_loop(0, _DGPT, fire, 0)

    def drain(g, carry):
        off = pl.multiple_of(g * G, G)
        pltpu.make_async_copy(
            w_loc.at[pl.ds(off, G)], deg_sh.at[dst_loc.at[g, 0]], sem).wait()
        return carry

    lax.fori_loop(0, _DGPT, drain, 0)

    @pl.when(t < _DTAIL)
    def _():
        gt = NC * NS * _DGPT + t
        pltpu.sync_copy(w_hbm.at[pl.ds(gt * G, G)], w_loc.at[pl.ds(0, G)])
        pltpu.sync_copy(dst3_hbm.at[pl.ds(gt, 1)], dst_loc.at[pl.ds(0, 1)])
        pltpu.async_copy(
            w_loc.at[pl.ds(0, G)], deg_sh.at[dst_loc.at[0, 0]], sem, add=True)
        pltpu.make_async_copy(
            w_loc.at[pl.ds(0, G)], deg_sh.at[dst_loc.at[0, 0]], sem).wait()

    plsc.subcore_barrier()
    pltpu.sync_copy(deg_sh.at[pl.ds(s * 640, 640)], out_hbm.at[c, pl.ds(s * 640, 640)])


_ROWS_PER_TILE = NPAD // NS
_ZROWS = 128
_GPT = NGROUPS // NS
_TAIL = NGROUPS - _GPT * NS
_CH = 39
_NCH = _GPT // _CH


@functools.partial(
    pl.kernel,
    out_type=[
        jax.ShapeDtypeStruct((NPAD, HALF), jnp.float32),
        jax.ShapeDtypeStruct((NPAD, HALF), jnp.float32),
    ],
    mesh=_sc_mesh,
    scratch_types=[
        pltpu.VMEM((_CH * G,), jnp.int32),
        pltpu.VMEM((_CH, 1, G), jnp.int32),
        pltpu.VMEM((_CH * G,), jnp.float32),
        pltpu.VMEM((G, HALF), jnp.float32),
        pltpu.VMEM((G, HALF), jnp.float32),
        pltpu.VMEM_SHARED((NPAD, HALF), jnp.float32),
        pltpu.SemaphoreType.DMA,
        pltpu.SemaphoreType.DMA,
        pltpu.SemaphoreType.DMA,
        pltpu.SemaphoreType.DMA,
    ],
)
def _agg_kernel(hp2n, src_hbm, dst3_hbm, w_hbm, outA, outB,
                src_loc, dst_loc, w_loc, rows0, rows1, acc_sh,
                sem0, sem1, sem2, sem3):
    c = lax.axis_index("c")
    s = lax.axis_index("s")

    def zrow(r, carry):
        for v8 in range(HALF // 16):
            rows0[r, pl.ds(v8 * 16, 16)] = jnp.zeros((16,), jnp.float32)
        return carry

    lax.fori_loop(0, _ZROWS, zrow, 0)
    for k5 in range(_ROWS_PER_TILE // _ZROWS):
        pltpu.sync_copy(rows0, acc_sh.at[pl.ds(s * _ROWS_PER_TILE + k5 * _ZROWS, _ZROWS)])
    plsc.subcore_barrier()

    def gather(g, buf, sem):
        off = pl.multiple_of(g * G, G)
        return pltpu.async_copy(
            hp2n.at[src_loc.at[pl.ds(off, G)]], buf, sem)

    def gwait(g, buf, sem):
        off = pl.multiple_of(g * G, G)
        pltpu.make_async_copy(
            hp2n.at[src_loc.at[pl.ds(off, G)]], buf, sem).wait()

    def scale(g, buf):
        def edge32(eo, carry2):
            for half in range(2):
                wv16 = w_loc[pl.ds(pl.multiple_of(g * G + eo * 32 + half * 16, 16), 16)]
                for lane in range(16):
                    sw = wv16[lane]
                    e = eo * 32 + half * 16 + lane
                    for v8 in range(HALF // 16):
                        buf[e, pl.ds(v8 * 16, 16)] = buf[e, pl.ds(v8 * 16, 16)] * sw
            return carry2

        lax.fori_loop(0, G // 32, edge32, 0)

    def scat_start(g, buf, sem):
        pltpu.async_copy(buf, acc_sh.at[dst_loc.at[g, 0]], sem, add=True)

    def scat_wait(g, buf, sem):
        pltpu.make_async_copy(buf, acc_sh.at[dst_loc.at[g, 0]], sem).wait()

    def stage(k):
        ebase = pl.multiple_of(s * (_GPT * G) + k * (_CH * G), G)
        pltpu.sync_copy(src_hbm.at[pl.ds(ebase, _CH * G)], src_loc)
        pltpu.sync_copy(dst3_hbm.at[pl.ds(s * _GPT + k * _CH, _CH)], dst_loc)
        pltpu.sync_copy(w_hbm.at[pl.ds(ebase, _CH * G)], w_loc)

        def tf(i, carry):
            v = src_loc[pl.ds(i * 16, 16)]
            src_loc[pl.ds(i * 16, 16)] = v * 2 + c
            return carry

        lax.fori_loop(0, (_CH * G) // 16, tf, 0)

    def chunk(k, carry):
        stage(k)
        gather(0, rows0, sem0)
        gather(1, rows1, sem1)

        def pair(p, carry2):
            g0 = 2 * p
            g1 = g0 + 1
            gwait(g0, rows0, sem0)
            scale(g0, rows0)
            scat_start(g0, rows0, sem2)
            gwait(g1, rows1, sem1)

            @pl.when(p < _CH // 2 - 1)
            def _():
                scat_wait(g0, rows0, sem2)
                gather(g0 + 2, rows0, sem0)

            scale(g1, rows1)
            pltpu.sync_copy(rows1, acc_sh.at[dst_loc.at[g1, 0]], add=True)

            @pl.when(p < _CH // 2 - 1)
            def _():
                gather(g1 + 2, rows1, sem1)

            @pl.when(p == _CH // 2 - 1)
            def _():
                scat_wait(g0, rows0, sem2)

            return carry2

        lax.fori_loop(0, _CH // 2, pair, 0)
        return carry

    lax.fori_loop(0, _NCH, chunk, 0)

    @pl.when(s < _TAIL)
    def _():
        gt = NS * _GPT + s
        pltpu.sync_copy(src_hbm.at[pl.ds(gt * G, G)], src_loc.at[pl.ds(0, G)])
        pltpu.sync_copy(dst3_hbm.at[pl.ds(gt, 1)], dst_loc.at[pl.ds(0, 1)])
        pltpu.sync_copy(w_hbm.at[pl.ds(gt * G, G)], w_loc.at[pl.ds(0, G)])

        def tf2(i, carry):
            v = src_loc[pl.ds(i * 16, 16)]
            src_loc[pl.ds(i * 16, 16)] = v * 2 + c
            return carry

        lax.fori_loop(0, G // 16, tf2, 0)
        gather(0, rows0, sem0)
        gwait(0, rows0, sem0)
        scale(0, rows0)
        scat_start(0, rows0, sem2)
        scat_wait(0, rows0, sem2)

    plsc.subcore_barrier()

    for k5 in range(_ROWS_PER_TILE // _ZROWS):
        r0 = s * _ROWS_PER_TILE + k5 * _ZROWS

        @pl.when(c == 0)
        def _():
            pltpu.sync_copy(acc_sh.at[pl.ds(r0, _ZROWS)], outA.at[pl.ds(r0, _ZROWS)])

        @pl.when(c == 1)
        def _():
            pltpu.sync_copy(acc_sh.at[pl.ds(r0, _ZROWS)], outB.at[pl.ds(r0, _ZROWS)])


_RB = 1000


def _k1_body(deg_ref, x_ref, w1_ref, hp_ref, dis_ref):
    dis = lax.rsqrt(deg_ref[...] + 1.0)
    h = jnp.dot(x_ref[...], w1_ref[...], preferred_element_type=jnp.float32)
    hp_ref[...] = h * dis
    dis_ref[...] = dis


def _tc_prep(deg_col, x, W1):
    grid = (N // _RB,)
    return pl.pallas_call(
        _k1_body,
        grid=grid,
        in_specs=[
            pl.BlockSpec((_RB, 1), lambda i: (i, 0)),
            pl.BlockSpec((_RB, IN_CH), lambda i: (i, 0)),
            pl.BlockSpec((IN_CH, HID), lambda i: (0, 0)),
        ],
        out_specs=[
            pl.BlockSpec((_RB, HID), lambda i: (i, 0)),
            pl.BlockSpec((_RB, 1), lambda i: (i, 0)),
        ],
        out_shape=[
            jax.ShapeDtypeStruct((N, HID), jnp.float32),
            jax.ShapeDtypeStruct((N, 1), jnp.float32),
        ],
    )(deg_col, x, W1)


def _k2_body(aa_ref, ab_ref, hp_ref, dis_ref, b_ref, w2_ref, o_ref):
    dis = dis_ref[...]
    z = jnp.concatenate([aa_ref[...], ab_ref[...]], axis=1) + hp_ref[...]
    z = jnp.maximum(z * dis + b_ref[...], 0.0)
    h2 = jnp.dot(z, w2_ref[...], preferred_element_type=jnp.float32)
    o_ref[...] = h2 * dis


def _tc_mid(accA, accB, hp, dis, b1, W2):
    grid = (N // _RB,)
    row = lambda i: (i, 0)
    return pl.pallas_call(
        _k2_body,
        grid=grid,
        in_specs=[
            pl.BlockSpec((_RB, HALF), row),
            pl.BlockSpec((_RB, HALF), row),
            pl.BlockSpec((_RB, HID), row),
            pl.BlockSpec((_RB, 1), row),
            pl.BlockSpec((1, HID), lambda i: (0, 0)),
            pl.BlockSpec((HID, HID), lambda i: (0, 0)),
        ],
        out_specs=pl.BlockSpec((_RB, HID), row),
        out_shape=jax.ShapeDtypeStruct((N, HID), jnp.float32),
    )(accA, accB, hp, dis, b1, W2)


def _k3_body(aa_ref, ab_ref, hp_ref, dis_ref, b_ref, wr_ref, br_ref,
             h2_ref, resc_ref):
    dis = dis_ref[...]
    z = jnp.concatenate([aa_ref[...], ab_ref[...]], axis=1) + hp_ref[...]
    z = jnp.maximum(z * dis + b_ref[...], 0.0)
    h2_ref[...] = z
    r = jnp.dot(z, wr_ref[...], preferred_element_type=jnp.float32) + br_ref[...]
    resc_ref[...] = 1.0 / (1.0 + jnp.exp(-r))


def _tc_final(accA, accB, hp2, dis, b2, Wr, br):
    grid = (N // _RB,)
    row = lambda i: (i, 0)
    return pl.pallas_call(
        _k3_body,
        grid=grid,
        in_specs=[
            pl.BlockSpec((_RB, HALF), row),
            pl.BlockSpec((_RB, HALF), row),
            pl.BlockSpec((_RB, HID), row),
            pl.BlockSpec((_RB, 1), row),
            pl.BlockSpec((1, HID), lambda i: (0, 0)),
            pl.BlockSpec((HID, 1), lambda i: (0, 0)),
            pl.BlockSpec((1, 1), lambda i: (0, 0)),
        ],
        out_specs=[
            pl.BlockSpec((_RB, HID), row),
            pl.BlockSpec((_RB, 1), row),
        ],
        out_shape=[
            jax.ShapeDtypeStruct((N, HID), jnp.float32),
            jax.ShapeDtypeStruct((N, 1), jnp.float32),
        ],
    )(accA, accB, hp2, dis, b2, Wr, br)


_HRB = 400


def _k4_body(h2_ref, wn_ref, m_ref, s_ref):
    i = pl.program_id(0)
    tile = jnp.dot(h2_ref[...], wn_ref[...], preferred_element_type=jnp.float32)
    tmax = jnp.max(tile, axis=0, keepdims=True)

    @pl.when(i == 0)
    def _():
        m_ref[...] = tmax
        s_ref[...] = jnp.sum(jnp.exp(tile - tmax), axis=0, keepdims=True)

    @pl.when(i > 0)
    def _():
        m_old = m_ref[...]
        m_new = jnp.maximum(m_old, tmax)
        s_ref[...] = s_ref[...] * jnp.exp(m_old - m_new) + jnp.sum(
            jnp.exp(tile - m_new), axis=0, keepdims=True)
        m_ref[...] = m_new


def _tc_head_stats(h2, Wn):
    grid = (N // _HRB,)
    return pl.pallas_call(
        _k4_body,
        grid=grid,
        in_specs=[
            pl.BlockSpec((_HRB, HID), lambda i: (i, 0)),
            pl.BlockSpec((HID, N), lambda i: (0, 0)),
        ],
        out_specs=[
            pl.BlockSpec((1, N), lambda i: (0, 0)),
            pl.BlockSpec((1, N), lambda i: (0, 0)),
        ],
        out_shape=[
            jax.ShapeDtypeStruct((1, N), jnp.float32),
            jax.ShapeDtypeStruct((1, N), jnp.float32),
        ],
    )(h2, Wn)


def _k5_body(h2_ref, wn_ref, m_ref, s_ref, out_ref):
    tile = jnp.dot(h2_ref[...], wn_ref[...], preferred_element_type=jnp.float32)
    out_ref[...] = jnp.exp(tile - m_ref[...]) * (1.0 / s_ref[...])


def _tc_norm(h2, Wn, m, s):
    grid = (N // _HRB,)
    return pl.pallas_call(
        _k5_body,
        grid=grid,
        in_specs=[
            pl.BlockSpec((_HRB, HID), lambda i: (i, 0)),
            pl.BlockSpec((HID, N), lambda i: (0, 0)),
            pl.BlockSpec((1, N), lambda i: (0, 0)),
            pl.BlockSpec((1, N), lambda i: (0, 0)),
        ],
        out_specs=pl.BlockSpec((_HRB, N), lambda i: (i, 0)),
        out_shape=jax.ShapeDtypeStruct((N, N), jnp.float32),
    )(h2, Wn, m, s)


def kernel(x, edge_index, edge_weight, W1, b1, W2, b2, Wn, bn, Wr, br):
    src = edge_index[0]
    dst = edge_index[1]
    dst3 = dst.reshape(NGROUPS, 1, G)

    deg2 = _deg_kernel(dst3, edge_weight)
    deg_col = (deg2[0, :N] + deg2[1, :N]).reshape(N, 1)

    hp, dis = _tc_prep(deg_col, x, W1)
    accA, accB = _agg_kernel(hp.reshape(2 * N, HALF), src, dst3, edge_weight)
    accA = accA[:N]
    accB = accB[:N]
    hp2 = _tc_mid(accA, accB, hp, dis, b1.reshape(1, HID), W2)
    acc2A, acc2B = _agg_kernel(hp2.reshape(2 * N, HALF), src, dst3, edge_weight)
    acc2A = acc2A[:N]
    acc2B = acc2B[:N]
    h2, rescue = _tc_final(acc2A, acc2B, hp2, dis,
                           b2.reshape(1, HID), Wr, br.reshape(1, 1))

    m, s = _tc_head_stats(h2, Wn)
    node_selector = _tc_norm(h2, Wn, m, s)
    return node_selector, rescue.reshape(N)
